# Initial kernel scaffold; baseline (speedup 1.0000x reference)
import jax
import jax.numpy as jnp
from jax.experimental import pallas as pl


def kernel(batch, x, edge_index, edge_attr, W_edge, b_edge, W1, b1, Wa, ba, Wb, bb):
    raise NotImplementedError("write your pallas kernel here")


# speedup vs baseline: 2.4769x; 2.4769x over previous
import functools

import jax
import jax.numpy as jnp
from jax import lax
from jax.experimental import pallas as pl
from jax.experimental.pallas import tpu as pltpu
from jax.experimental.pallas import tpu_sc as plsc

CH = 128
NW = 32
LANES = 16


def _proj_tc(edge_attr, W_edge, b_edge):
    E, DE = edge_attr.shape
    D = W_edge.shape[1]
    BLK = 4000

    def body(ea_ref, w_ref, b_ref, out_ref):
        out_ref[...] = jnp.dot(ea_ref[...], w_ref[...],
                               preferred_element_type=jnp.float32,
                               precision=lax.Precision.HIGHEST) + b_ref[...]

    return pl.pallas_call(
        body,
        grid=(E // BLK,),
        in_specs=[
            pl.BlockSpec((BLK, DE), lambda i: (i, 0)),
            pl.BlockSpec((DE, D), lambda i: (0, 0)),
            pl.BlockSpec((1, D), lambda i: (0, 0)),
        ],
        out_specs=pl.BlockSpec((BLK, D), lambda i: (i, 0)),
        out_shape=jax.ShapeDtypeStruct((E, D), jnp.float32),
    )(edge_attr, W_edge, b_edge.reshape(1, D))


def _sc_message(x, zeros_nd, proj, src, dst):
    N, D = x.shape
    E = src.shape[0]
    nchunk = E // CH
    cpw = (nchunk + NW - 1) // NW
    rows_per_tile = ((N + 15 * 8) // (16 * 8)) * 8
    mesh = plsc.VectorSubcoreMesh(core_axis_name="c", subcore_axis_name="s")

    @functools.partial(
        pl.kernel,
        out_type=jax.ShapeDtypeStruct((2, N, D), jnp.float32),
        mesh=mesh,
        name="sc_message",
        scratch_types=[
            pltpu.VMEM_SHARED((N, D), jnp.float32),
            pltpu.VMEM((CH,), jnp.int32),
            pltpu.VMEM((CH,), jnp.int32),
            pltpu.VMEM((CH, D), jnp.float32),
            pltpu.VMEM((CH, D), jnp.float32),
            pltpu.SemaphoreType.DMA,
        ],
    )
    def sc_message_k(x_hbm, zeros_hbm, proj_hbm, src_hbm, dst_hbm, out_hbm,
          agg_sh, sidx, didx, xrows, prows, sem):
        c = lax.axis_index("c")
        s = lax.axis_index("s")
        wid = c * 16 + s

        rb = pl.multiple_of(jnp.minimum(s * rows_per_tile, N - rows_per_tile), 8)
        pltpu.sync_copy(zeros_hbm.at[pl.ds(rb, rows_per_tile)],
                        agg_sh.at[pl.ds(rb, rows_per_tile)])
        plsc.subcore_barrier()

        def chunk(kk, carry):
            cidx = kk * NW + wid

            @pl.when(cidx < nchunk)
            def _():
                gb = cidx * CH
                pltpu.sync_copy(src_hbm.at[pl.ds(gb, CH)], sidx)
                pltpu.sync_copy(dst_hbm.at[pl.ds(gb, CH)], didx)
                gat = pltpu.async_copy(x_hbm.at[sidx], xrows, sem)
                pltpu.sync_copy(proj_hbm.at[pl.ds(gb, CH)], prows)
                gat.wait()

                def row(r, c2):
                    for j in range(D // LANES):
                        sl = pl.ds(j * LANES, LANES)
                        xrows[r, sl] = jnp.maximum(xrows[r, sl] + prows[r, sl], 0.0)
                    return c2
                lax.fori_loop(0, CH, row, None)
                pltpu.sync_copy(xrows, agg_sh.at[didx], add=True)
            return carry
        lax.fori_loop(0, cpw, chunk, None)

        plsc.subcore_barrier()
        pltpu.sync_copy(agg_sh.at[pl.ds(rb, rows_per_tile)],
                        out_hbm.at[c, pl.ds(rb, rows_per_tile)])


    return sc_message_k(x, zeros_nd, proj, src, dst)


def _node_tc(x, agg0, agg1, W1, b1, WaCat):
    N, D = x.shape
    BLK = 1000

    def body(x_ref, a0_ref, a1_ref, w1_ref, b1_ref, wa_ref, out_ref):
        z = x_ref[...] + a0_ref[...] + a1_ref[...]
        ne = jnp.maximum(
            jnp.dot(z, w1_ref[...], preferred_element_type=jnp.float32,
                    precision=lax.Precision.HIGHEST)
            + b1_ref[...], 0.0)
        out_ref[...] = jnp.dot(ne, wa_ref[...], preferred_element_type=jnp.float32,
                               precision=lax.Precision.HIGHEST)

    return pl.pallas_call(
        body,
        grid=(N // BLK,),
        in_specs=[
            pl.BlockSpec((BLK, D), lambda i: (i, 0)),
            pl.BlockSpec((BLK, D), lambda i: (i, 0)),
            pl.BlockSpec((BLK, D), lambda i: (i, 0)),
            pl.BlockSpec((D, D), lambda i: (0, 0)),
            pl.BlockSpec((1, D), lambda i: (0, 0)),
            pl.BlockSpec((D, D), lambda i: (0, 0)),
        ],
        out_specs=pl.BlockSpec((BLK, D), lambda i: (i, 0)),
        out_shape=jax.ShapeDtypeStruct((N, D), jnp.float32),
    )(x, agg0, agg1, W1, b1.reshape(1, D), WaCat)


def _sc_edge(P, src, dst, ba, wb):
    N, D2 = P.shape
    H = D2 // 2
    E = src.shape[0]
    nchunk = E // CH
    cpw = (nchunk + NW - 1) // NW
    mesh = plsc.VectorSubcoreMesh(core_axis_name="c", subcore_axis_name="s")

    @functools.partial(
        pl.kernel,
        out_type=jax.ShapeDtypeStruct((E, LANES), jnp.float32),
        mesh=mesh,
        name="sc_edge",
        scratch_types=[
            pltpu.VMEM((CH,), jnp.int32),
            pltpu.VMEM((CH,), jnp.int32),
            pltpu.VMEM((CH, D2), jnp.float32),
            pltpu.VMEM((CH, D2), jnp.float32),
            pltpu.VMEM((CH, LANES), jnp.float32),
            pltpu.VMEM((H,), jnp.float32),
            pltpu.VMEM((H,), jnp.float32),
            pltpu.SemaphoreType.DMA,
            pltpu.SemaphoreType.DMA,
        ],
    )
    def sc_edge_k(p_hbm, src_hbm, dst_hbm, ba_hbm, wb_hbm, out_hbm,
          sidx, didx, r1, r2, acc, bav, wbv, sem1, sem2):
        wid = lax.axis_index("c") * 16 + lax.axis_index("s")
        pltpu.sync_copy(ba_hbm, bav)
        pltpu.sync_copy(wb_hbm, wbv)

        def chunk(kk, carry):
            cidx = kk * NW + wid

            @pl.when(cidx < nchunk)
            def _():
                gb = cidx * CH
                pltpu.sync_copy(src_hbm.at[pl.ds(gb, CH)], sidx)
                pltpu.sync_copy(dst_hbm.at[pl.ds(gb, CH)], didx)
                g1 = pltpu.async_copy(p_hbm.at[sidx], r1, sem1)
                g2 = pltpu.async_copy(p_hbm.at[didx], r2, sem2)
                g1.wait()
                g2.wait()

                def row(e, c2):
                    a = jnp.zeros((LANES,), jnp.float32)
                    for j in range(H // LANES):
                        sl = pl.ds(j * LANES, LANES)
                        g = (r1[e, sl] + r2[e, pl.ds(H + j * LANES, LANES)]
                             + bav[pl.ds(j * LANES, LANES)])
                        a = a + jnp.maximum(g, 0.0) * wbv[pl.ds(j * LANES, LANES)]
                    acc[e, :] = a
                    return c2
                lax.fori_loop(0, CH, row, None)
                pltpu.sync_copy(acc, out_hbm.at[pl.ds(gb, CH)])
            return carry
        lax.fori_loop(0, cpw, chunk, None)

    return sc_edge_k(P, src, dst, ba, wb)


def _fold_tc(acc16, bb):
    E = acc16.shape[0]
    R = E // 8
    a_r = acc16.reshape(R, 128)
    fold = jnp.zeros((128, 8), jnp.float32)
    fold = fold.at[jnp.arange(128), jnp.arange(128) // 16].set(1.0)
    BLK = 4000

    def body(a_ref, f_ref, b_ref, out_ref):
        out_ref[...] = jnp.dot(a_ref[...], f_ref[...],
                               preferred_element_type=jnp.float32,
                               precision=lax.Precision.HIGHEST) + b_ref[...]

    out = pl.pallas_call(
        body,
        grid=(R // BLK,),
        in_specs=[
            pl.BlockSpec((BLK, 128), lambda i: (i, 0)),
            pl.BlockSpec((128, 8), lambda i: (0, 0)),
            pl.BlockSpec((1, 1), lambda i: (0, 0)),
        ],
        out_specs=pl.BlockSpec((BLK, 8), lambda i: (i, 0)),
        out_shape=jax.ShapeDtypeStruct((R, 8), jnp.float32),
    )(a_r, fold, bb.reshape(1, 1))
    return out.reshape(E, 1)


def kernel(batch, x, edge_index, edge_attr, W_edge, b_edge, W1, b1, Wa, ba, Wb, bb):
    del batch
    D = x.shape[1]
    src = edge_index[0]
    dst = edge_index[1]

    proj = _proj_tc(edge_attr, W_edge, b_edge)
    zeros_nd = jnp.zeros_like(x)
    aggs = _sc_message(x, zeros_nd, proj, src, dst)

    WaCat = jnp.concatenate([Wa[:D], Wa[D:]], axis=1)
    P = _node_tc(x, aggs[0], aggs[1], W1, b1, WaCat)

    acc16 = _sc_edge(P, src, dst, ba, Wb.reshape(-1))
    return _fold_tc(acc16, bb)
